# baseline (speedup 1.0000x reference)
import jax
import jax.numpy as jnp
from jax.experimental import pallas as pl

_DN = (((1,), (1,)), ((), ()))


def _router_body(x_ref, gw_ref, nw_ref, n_ref, o_ref):
    xb = x_ref[...]
    clean = jax.lax.dot_general(xb, gw_ref[...], _DN,
                                preferred_element_type=jnp.float32)
    raw = jax.lax.dot_general(xb, nw_ref[...], _DN,
                              preferred_element_type=jnp.float32)
    ns = jax.nn.softplus(raw)
    ew = clean + n_ref[...] * ns
    m1 = jnp.max(ew, axis=1, keepdims=True)
    is1 = ew == m1
    ew2 = jnp.where(is1, -jnp.inf, ew)
    m2 = jnp.max(ew2, axis=1, keepdims=True)
    is2 = ew2 == m2
    e2 = jnp.exp(m2 - m1)
    inv = 1.0 / (1.0 + e2)
    o_ref[...] = jnp.where(is1, inv, jnp.where(is2, e2 * inv, 0.0))


def kernel(x, gate_weights, noise_weights, noise):
    n_tokens, d_model = x.shape
    n_experts = gate_weights.shape[0]
    bt = 512
    return pl.pallas_call(
        _router_body,
        grid=(n_tokens // bt,),
        in_specs=[
            pl.BlockSpec((bt, d_model), lambda i: (i, 0)),
            pl.BlockSpec((n_experts, d_model), lambda i: (0, 0)),
            pl.BlockSpec((n_experts, d_model), lambda i: (0, 0)),
            pl.BlockSpec((bt, n_experts), lambda i: (i, 0)),
        ],
        out_specs=pl.BlockSpec((bt, n_experts), lambda i: (i, 0)),
        out_shape=jax.ShapeDtypeStruct((n_tokens, n_experts), jnp.float32),
    )(x, gate_weights, noise_weights, noise)


# speedup vs baseline: 1.3244x; 1.3244x over previous
import jax
import jax.numpy as jnp
from jax.experimental import pallas as pl

_DN = (((1,), (1,)), ((), ()))


def _router_body(x_ref, gw_ref, nw_ref, n_ref, o_ref):
    xb = x_ref[...]
    clean = jax.lax.dot_general(xb, gw_ref[...], _DN,
                                preferred_element_type=jnp.float32)
    raw = jax.lax.dot_general(xb, nw_ref[...], _DN,
                              preferred_element_type=jnp.float32)
    ns = jax.nn.softplus(raw)
    ew = clean + n_ref[...] * ns
    m1 = jnp.max(ew, axis=1, keepdims=True)
    is1 = ew == m1
    ew2 = jnp.where(is1, -jnp.inf, ew)
    m2 = jnp.max(ew2, axis=1, keepdims=True)
    is2 = ew2 == m2
    e2 = jnp.exp(m2 - m1)
    inv = 1.0 / (1.0 + e2)
    o_ref[...] = jnp.where(is1, inv, jnp.where(is2, e2 * inv, 0.0))


def kernel(x, gate_weights, noise_weights, noise):
    n_tokens, d_model = x.shape
    n_experts = gate_weights.shape[0]
    bt = 2048
    return pl.pallas_call(
        _router_body,
        grid=(n_tokens // bt,),
        in_specs=[
            pl.BlockSpec((bt, d_model), lambda i: (i, 0)),
            pl.BlockSpec((n_experts, d_model), lambda i: (0, 0)),
            pl.BlockSpec((n_experts, d_model), lambda i: (0, 0)),
            pl.BlockSpec((bt, n_experts), lambda i: (i, 0)),
        ],
        out_specs=pl.BlockSpec((bt, n_experts), lambda i: (i, 0)),
        out_shape=jax.ShapeDtypeStruct((n_tokens, n_experts), jnp.float32),
    )(x, gate_weights, noise_weights, noise)


# speedup vs baseline: 1.3709x; 1.0351x over previous
import jax
import jax.numpy as jnp
from jax.experimental import pallas as pl

_DN = (((1,), (1,)), ((), ()))


def _router_body(x_ref, gw_ref, nw_ref, n_ref, o_ref):
    xb = x_ref[...]
    clean = jax.lax.dot_general(xb, gw_ref[...], _DN,
                                preferred_element_type=jnp.float32)
    raw = jax.lax.dot_general(xb, nw_ref[...], _DN,
                              preferred_element_type=jnp.float32)
    ns = jax.nn.softplus(raw)
    ew = clean + n_ref[...] * ns
    m1 = jnp.max(ew, axis=1, keepdims=True)
    is1 = ew == m1
    ew2 = jnp.where(is1, -jnp.inf, ew)
    m2 = jnp.max(ew2, axis=1, keepdims=True)
    is2 = ew2 == m2
    e2 = jnp.exp(m2 - m1)
    inv = 1.0 / (1.0 + e2)
    o_ref[...] = jnp.where(is1, inv, jnp.where(is2, e2 * inv, 0.0))


def kernel(x, gate_weights, noise_weights, noise):
    n_tokens, d_model = x.shape
    n_experts = gate_weights.shape[0]
    bt = 4096
    return pl.pallas_call(
        _router_body,
        grid=(n_tokens // bt,),
        in_specs=[
            pl.BlockSpec((bt, d_model), lambda i: (i, 0)),
            pl.BlockSpec((n_experts, d_model), lambda i: (0, 0)),
            pl.BlockSpec((n_experts, d_model), lambda i: (0, 0)),
            pl.BlockSpec((bt, n_experts), lambda i: (i, 0)),
        ],
        out_specs=pl.BlockSpec((bt, n_experts), lambda i: (i, 0)),
        out_shape=jax.ShapeDtypeStruct((n_tokens, n_experts), jnp.float32),
    )(x, gate_weights, noise_weights, noise)
